# baseline (speedup 1.0000x reference)
import functools
import jax
import jax.numpy as jnp
from jax import lax
from jax.experimental import pallas as pl
from jax.experimental.pallas import tpu as pltpu
from jax.experimental.pallas import tpu_sc as plsc

_RB = 3200
_NC, _NS = 2, 16
_EB = 1024
_K = 128


def _sc_mesh():
  return plsc.VectorSubcoreMesh(core_axis_name="c", subcore_axis_name="s",
                                num_cores=_NC, num_subcores=_NS)


_RROWS = 10240


def _sc_pass(src, dst, tables, npad):
  e = dst.shape[0]
  ntab = len(tables)
  nout = max(ntab, 1)
  nrange = -(-npad // _RROWS)
  npr = -(-nrange // _NC)
  ept = e // _NS
  nfull = ept // _EB
  tail = ept - nfull * _EB
  ntailf = (tail + _K - 1) // _K
  tailpad = ntailf * _K - tail
  stripe = _RROWS // _NS
  zrows = 128
  assert stripe % zrows == 0

  scratch = [
      pltpu.VMEM_SHARED((_RROWS + 8, 128), jnp.float32),
      pltpu.VMEM((_EB,), jnp.int32),
      pltpu.VMEM((_EB,), jnp.int32),
      pltpu.VMEM((_EB + _K,), jnp.int32),
      pltpu.VMEM((zrows, 128), jnp.float32),
      pltpu.VMEM((_K, 128), jnp.float32),
      pltpu.SemaphoreType.DMA,
  ]

  @functools.partial(
      pl.kernel,
      out_type=jax.ShapeDtypeStruct((nout * npad, 128), jnp.float32),
      mesh=_sc_mesh(),
      scratch_types=scratch,
  )
  def k(*refs):
    dst_hbm, src_hbm = refs[0], refs[1]
    tab_hbm = refs[2:2 + ntab]
    out_hbm = refs[2 + ntab]
    slab, dstblk, srcblk, cdst, zbuf, rbuf, sem = refs[3 + ntab:]

    c = lax.axis_index("c")
    s = lax.axis_index("s")
    tile_e0 = s * ept

    def zrow(i, _):
      def zl(l, _):
        zbuf[i, pl.ds(l * 16, 16)] = jnp.zeros((16,), jnp.float32)
        return 0
      lax.fori_loop(0, 8, zl, 0)
      return 0
    lax.fori_loop(0, zrows, zrow, 0)
    if ntab == 0:
      def orow(i, _):
        def ol(l, _):
          rbuf[i, pl.ds(l * 16, 16)] = jnp.full((16,), 1.0, jnp.float32)
          return 0
        lax.fori_loop(0, 8, ol, 0)
        return 0
      lax.fori_loop(0, _K, orow, 0)

    def remap(base, blk_len):
      def step(i, _):
        d = dstblk[pl.ds(i * 16, 16)]
        ld = d - base
        u = plsc.bitcast(ld, jnp.uint32)
        t = jnp.minimum(u, jnp.uint32(_RROWS))
        cdst[pl.ds(i * 16, 16)] = plsc.bitcast(t, jnp.int32)
        return 0
      lax.fori_loop(0, blk_len // 16, step, 0)

    def fire(t, nf):
      def f(j, _):
        if ntab:
          pltpu.async_copy(tab_hbm[t].at[srcblk.at[pl.ds(j * _K, _K)]],
                           rbuf, sem).wait()
        pltpu.sync_copy(rbuf, slab.at[cdst.at[pl.ds(j * _K, _K)]], add=True)
        return 0
      lax.fori_loop(0, nf, f, 0)

    def do_pair(t, r):
      base = (c * npr + r) * _RROWS

      def zcp(i, _):
        pltpu.sync_copy(zbuf, slab.at[pl.ds(s * stripe + i * zrows, zrows)])
        return 0
      lax.fori_loop(0, stripe // zrows, zcp, 0)
      plsc.subcore_barrier()

      def blk(b, _):
        off = tile_e0 + b * _EB
        pltpu.sync_copy(dst_hbm.at[pl.ds(off, _EB)], dstblk.at[pl.ds(0, _EB)])
        if ntab:
          pltpu.sync_copy(src_hbm.at[pl.ds(off, _EB)],
                          srcblk.at[pl.ds(0, _EB)])
        remap(base, _EB)
        fire(t, _EB // _K)
        return 0
      lax.fori_loop(0, nfull, blk, 0)
      if tail:
        off = tile_e0 + nfull * _EB
        pltpu.sync_copy(dst_hbm.at[pl.ds(off, tail)], dstblk.at[pl.ds(0, tail)])
        if ntab:
          pltpu.sync_copy(src_hbm.at[pl.ds(off, tail)],
                          srcblk.at[pl.ds(0, tail)])
        remap(base, tail)

        def padv(u, _):
          cdst[pl.ds(tail + u * 16, 16)] = jnp.full((16,), _RROWS, jnp.int32)
          return 0
        lax.fori_loop(0, tailpad // 16, padv, 0)
        fire(t, ntailf)

      plsc.subcore_barrier()

      @pl.when(base < npad)
      def _():
        pltpu.sync_copy(
            slab.at[pl.ds(s * stripe, stripe)],
            out_hbm.at[pl.ds(t * npad + base + s * stripe, stripe)])
      plsc.subcore_barrier()

    for t in range(nout):
      def rloop(r, _, t=t):
        do_pair(t, r)
        return 0
      lax.fori_loop(0, npr, rloop, 0)

  return k(dst, src, *tables)


def _tc_norm_scale(deg128, xt128, npad):
  nb = npad // _RB

  def body(dref, xref, oref):
    dinv = lax.rsqrt(dref[:, 0:1] + 1.0)
    oref[...] = xref[...] * dinv

  return pl.pallas_call(
      body,
      grid=(nb,),
      in_specs=[
          pl.BlockSpec((_RB, 128), lambda g: (g, 0)),
          pl.BlockSpec((_RB, 128), lambda g: (g, 0)),
      ],
      out_specs=pl.BlockSpec((_RB, 128), lambda g: (g, 0)),
      out_shape=jax.ShapeDtypeStruct((npad, 128), jnp.float32),
  )(deg128, xt128)


def _tc_hidden(praw, xp128, w1e, b1t, npad, hw):
  nb = npad // _RB

  def body(pref, xref, wref, bref, *orefs):
    xp = xref[...]
    dinv = xp[:, 24:25]
    p = dinv * (pref[...] + xp)
    h = jnp.maximum(jnp.dot(p, wref[...],
                            preferred_element_type=jnp.float32) + bref[...],
                    0.0)
    hp = dinv * h
    for t in range(4):
      orefs[t][...] = hp[:, t * 128:(t + 1) * 128]

  return pl.pallas_call(
      body,
      grid=(nb,),
      in_specs=[
          pl.BlockSpec((_RB, 128), lambda g: (g, 0)),
          pl.BlockSpec((_RB, 128), lambda g: (g, 0)),
          pl.BlockSpec((128, hw), lambda g: (0, 0)),
          pl.BlockSpec((1, hw), lambda g: (0, 0)),
      ],
      out_specs=[pl.BlockSpec((_RB, 128), lambda g: (g, 0))] * 4,
      out_shape=[jax.ShapeDtypeStruct((npad, 128), jnp.float32)] * 4,
  )(praw, xp128, w1e, b1t)


def _tc_out(qflat, hps, xp128, w2, b2r, wiht, bihr, whht, bhhr, wfct, bfcr,
            n, npad, t_steps, h_gcn, h_gru):
  nb = npad // _RB
  npb = npad // _RB

  def body(q0, q1, q2, q3, h0, h1, h2, h3, xref, w2ref, b2ref, wihref,
           bihref, whhref, bhhref, wfcref, bfcref, pooled_ref, out_ref):
    g = pl.program_id(0)
    dinv = xref[:, 24:25]
    qr = jnp.concatenate([q0[...], q1[...], q2[...], q3[...]], axis=1)
    hp = jnp.concatenate([h0[...], h1[...], h2[...], h3[...]], axis=1)
    q = dinv * (qr + hp)
    rowid = g * _RB + lax.broadcasted_iota(jnp.int32, (_RB, h_gcn), 0)
    mask = jnp.where(rowid < n, 1.0, 0.0)
    parts = []
    for t in range(t_steps):
      zt = jnp.maximum(
          jnp.dot(q[:, t * h_gcn:(t + 1) * h_gcn], w2ref[...],
                  preferred_element_type=jnp.float32) + b2ref[...], 0.0)
      parts.append(jnp.sum(zt * mask, axis=0, keepdims=True))
    contrib = jnp.concatenate(parts, axis=0)

    @pl.when(g == 0)
    def _():
      pooled_ref[...] = jnp.zeros_like(pooled_ref)
      out_ref[...] = jnp.zeros_like(out_ref)

    pooled_ref[...] += contrib

    @pl.when(g == nb - 1)
    def _():
      seq = pooled_ref[...] * (1.0 / n)
      h = jnp.zeros((1, h_gru), jnp.float32)
      for t in range(t_steps):
        st = seq[t:t + 1, :]
        gx = jnp.dot(st, wihref[...],
                     preferred_element_type=jnp.float32) + bihref[...]
        gh = jnp.dot(h, whhref[...],
                     preferred_element_type=jnp.float32) + bhhref[...]
        r = jax.nn.sigmoid(gx[:, :h_gru] + gh[:, :h_gru])
        z = jax.nn.sigmoid(gx[:, h_gru:2 * h_gru] + gh[:, h_gru:2 * h_gru])
        nn = jnp.tanh(gx[:, 2 * h_gru:] + r * gh[:, 2 * h_gru:])
        h = (1.0 - z) * nn + z * h
      out_ref[...] = jnp.dot(h, wfcref[...],
                             preferred_element_type=jnp.float32) + bfcref[...]

  qspecs = [pl.BlockSpec((_RB, 128), functools.partial(
      lambda g, kk: (kk * npb + g, 0), kk=kk)) for kk in range(4)]
  hspecs = [pl.BlockSpec((_RB, 128), lambda g: (g, 0)) for _ in range(4)]
  pooled, out = pl.pallas_call(
      body,
      grid=(nb,),
      in_specs=qspecs + hspecs + [
          pl.BlockSpec((_RB, 128), lambda g: (g, 0)),
          pl.BlockSpec((h_gcn, h_gcn), lambda g: (0, 0)),
          pl.BlockSpec((1, h_gcn), lambda g: (0, 0)),
          pl.BlockSpec((h_gcn, 3 * h_gru), lambda g: (0, 0)),
          pl.BlockSpec((1, 3 * h_gru), lambda g: (0, 0)),
          pl.BlockSpec((h_gru, 3 * h_gru), lambda g: (0, 0)),
          pl.BlockSpec((1, 3 * h_gru), lambda g: (0, 0)),
          pl.BlockSpec((h_gru, 128), lambda g: (0, 0)),
          pl.BlockSpec((1, 128), lambda g: (0, 0)),
      ],
      out_specs=[
          pl.BlockSpec((t_steps, h_gcn), lambda g: (0, 0)),
          pl.BlockSpec((1, 128), lambda g: (0, 0)),
      ],
      out_shape=[
          jax.ShapeDtypeStruct((t_steps, h_gcn), jnp.float32),
          jax.ShapeDtypeStruct((1, 128), jnp.float32),
      ],
  )(qflat, qflat, qflat, qflat, *hps, xp128, w2, b2r, wiht, bihr, whht,
    bhhr, wfct, bfcr)
  del pooled
  return out


def kernel(x, edge_index, W1, b1, W2, b2, Wih, Whh, bih, bhh, Wfc, bfc):
  t_steps, n, d_in = x.shape
  h_gcn = W1.shape[1]
  h_gru = Whh.shape[1]
  hw = t_steps * h_gcn
  npad = ((n + _RB - 1) // _RB) * _RB
  src, dst = edge_index[0], edge_index[1]

  xt = jnp.transpose(x, (1, 0, 2)).reshape(n, t_steps * d_in)
  xt128 = jnp.zeros((npad, 128), jnp.float32)
  xt128 = xt128.at[:n, :t_steps * d_in].set(xt)
  xt128 = xt128.at[:n, 24].set(1.0)

  w1e = jnp.zeros((128, hw), jnp.float32)
  for t in range(t_steps):
    w1e = w1e.at[t * d_in:(t + 1) * d_in, t * h_gcn:(t + 1) * h_gcn].set(W1)
  b1t = jnp.tile(b1, (t_steps,)).reshape(1, hw)
  b2r = b2.reshape(1, h_gcn)
  wiht = Wih.T
  whht = Whh.T
  bihr = bih.reshape(1, 3 * h_gru)
  bhhr = bhh.reshape(1, 3 * h_gru)
  wfct = jnp.zeros((h_gru, 128), jnp.float32).at[:, :1].set(Wfc.T)
  bfcr = jnp.zeros((1, 128), jnp.float32).at[0, 0].set(bfc[0])

  deg128 = _sc_pass(src, dst, [], npad)
  xp128 = _tc_norm_scale(deg128, xt128, npad)
  praw = _sc_pass(src, dst, [xp128], npad)
  hps = _tc_hidden(praw, xp128, w1e, b1t, npad, hw)
  qflat = _sc_pass(src, dst, list(hps), npad)
  out = _tc_out(qflat, hps, xp128, W2, b2r, wiht, bihr, whht, bhhr, wfct,
                bfcr, n, npad, t_steps, h_gcn, h_gru)
  return out[0, 0]


# speedup vs baseline: 1.0164x; 1.0164x over previous
import functools
import jax
import jax.numpy as jnp
from jax import lax
from jax.experimental import pallas as pl
from jax.experimental.pallas import tpu as pltpu
from jax.experimental.pallas import tpu_sc as plsc

_RB = 3200
_NC, _NS = 2, 16
_EB = 2048
_K = 128


def _sc_mesh():
  return plsc.VectorSubcoreMesh(core_axis_name="c", subcore_axis_name="s",
                                num_cores=_NC, num_subcores=_NS)


_RROWS = 10240


def _sc_pass(src, dst, tables, npad):
  e = dst.shape[0]
  ntab = len(tables)
  nout = max(ntab, 1)
  nrange = -(-npad // _RROWS)
  npr = -(-nrange // _NC)
  ept = e // _NS
  nfull = ept // _EB
  tail = ept - nfull * _EB
  ntailf = (tail + _K - 1) // _K
  tailpad = ntailf * _K - tail
  stripe = _RROWS // _NS
  zrows = 128
  assert stripe % zrows == 0

  scratch = [
      pltpu.VMEM_SHARED((_RROWS + 8, 128), jnp.float32),
      pltpu.VMEM((_EB,), jnp.int32),
      pltpu.VMEM((_EB,), jnp.int32),
      pltpu.VMEM((_EB + _K,), jnp.int32),
      pltpu.VMEM((zrows, 128), jnp.float32),
      pltpu.VMEM((_K, 128), jnp.float32),
      pltpu.SemaphoreType.DMA,
  ]

  @functools.partial(
      pl.kernel,
      out_type=jax.ShapeDtypeStruct((nout * npad, 128), jnp.float32),
      mesh=_sc_mesh(),
      scratch_types=scratch,
  )
  def k(*refs):
    dst_hbm, src_hbm = refs[0], refs[1]
    tab_hbm = refs[2:2 + ntab]
    out_hbm = refs[2 + ntab]
    slab, dstblk, srcblk, cdst, zbuf, rbuf, sem = refs[3 + ntab:]

    c = lax.axis_index("c")
    s = lax.axis_index("s")
    tile_e0 = s * ept

    def zrow(i, _):
      def zl(l, _):
        zbuf[i, pl.ds(l * 16, 16)] = jnp.zeros((16,), jnp.float32)
        return 0
      lax.fori_loop(0, 8, zl, 0)
      return 0
    lax.fori_loop(0, zrows, zrow, 0)
    if ntab == 0:
      def orow(i, _):
        def ol(l, _):
          rbuf[i, pl.ds(l * 16, 16)] = jnp.full((16,), 1.0, jnp.float32)
          return 0
        lax.fori_loop(0, 8, ol, 0)
        return 0
      lax.fori_loop(0, _K, orow, 0)

    def remap(base, blk_len):
      def step(i, _):
        d = dstblk[pl.ds(i * 16, 16)]
        ld = d - base
        u = plsc.bitcast(ld, jnp.uint32)
        t = jnp.minimum(u, jnp.uint32(_RROWS))
        cdst[pl.ds(i * 16, 16)] = plsc.bitcast(t, jnp.int32)
        return 0
      lax.fori_loop(0, blk_len // 16, step, 0)

    def fire(t, nf):
      def f(j, _):
        if ntab:
          pltpu.async_copy(tab_hbm[t].at[srcblk.at[pl.ds(j * _K, _K)]],
                           rbuf, sem).wait()
        pltpu.sync_copy(rbuf, slab.at[cdst.at[pl.ds(j * _K, _K)]], add=True)
        return 0
      lax.fori_loop(0, nf, f, 0)

    def do_pair(t, r):
      base = (c * npr + r) * _RROWS

      def zcp(i, _):
        pltpu.sync_copy(zbuf, slab.at[pl.ds(s * stripe + i * zrows, zrows)])
        return 0
      lax.fori_loop(0, stripe // zrows, zcp, 0)
      plsc.subcore_barrier()

      def blk(b, _):
        off = tile_e0 + b * _EB
        pltpu.sync_copy(dst_hbm.at[pl.ds(off, _EB)], dstblk.at[pl.ds(0, _EB)])
        if ntab:
          pltpu.sync_copy(src_hbm.at[pl.ds(off, _EB)],
                          srcblk.at[pl.ds(0, _EB)])
        remap(base, _EB)
        fire(t, _EB // _K)
        return 0
      lax.fori_loop(0, nfull, blk, 0)
      if tail:
        off = tile_e0 + nfull * _EB
        pltpu.sync_copy(dst_hbm.at[pl.ds(off, tail)], dstblk.at[pl.ds(0, tail)])
        if ntab:
          pltpu.sync_copy(src_hbm.at[pl.ds(off, tail)],
                          srcblk.at[pl.ds(0, tail)])
        remap(base, tail)

        def padv(u, _):
          cdst[pl.ds(tail + u * 16, 16)] = jnp.full((16,), _RROWS, jnp.int32)
          return 0
        lax.fori_loop(0, tailpad // 16, padv, 0)
        fire(t, ntailf)

      plsc.subcore_barrier()

      @pl.when(base < npad)
      def _():
        pltpu.sync_copy(
            slab.at[pl.ds(s * stripe, stripe)],
            out_hbm.at[pl.ds(t * npad + base + s * stripe, stripe)])
      plsc.subcore_barrier()

    for t in range(nout):
      def rloop(r, _, t=t):
        do_pair(t, r)
        return 0
      lax.fori_loop(0, npr, rloop, 0)

  return k(dst, src, *tables)


def _tc_norm_scale(deg128, xt128, npad):
  nb = npad // _RB

  def body(dref, xref, oref):
    dinv = lax.rsqrt(dref[:, 0:1] + 1.0)
    oref[...] = xref[...] * dinv

  return pl.pallas_call(
      body,
      grid=(nb,),
      in_specs=[
          pl.BlockSpec((_RB, 128), lambda g: (g, 0)),
          pl.BlockSpec((_RB, 128), lambda g: (g, 0)),
      ],
      out_specs=pl.BlockSpec((_RB, 128), lambda g: (g, 0)),
      out_shape=jax.ShapeDtypeStruct((npad, 128), jnp.float32),
  )(deg128, xt128)


def _tc_hidden(praw, xp128, w1e, b1t, npad, hw):
  nb = npad // _RB

  def body(pref, xref, wref, bref, *orefs):
    xp = xref[...]
    dinv = xp[:, 24:25]
    p = dinv * (pref[...] + xp)
    h = jnp.maximum(jnp.dot(p, wref[...],
                            preferred_element_type=jnp.float32) + bref[...],
                    0.0)
    hp = dinv * h
    for t in range(4):
      orefs[t][...] = hp[:, t * 128:(t + 1) * 128]

  return pl.pallas_call(
      body,
      grid=(nb,),
      in_specs=[
          pl.BlockSpec((_RB, 128), lambda g: (g, 0)),
          pl.BlockSpec((_RB, 128), lambda g: (g, 0)),
          pl.BlockSpec((128, hw), lambda g: (0, 0)),
          pl.BlockSpec((1, hw), lambda g: (0, 0)),
      ],
      out_specs=[pl.BlockSpec((_RB, 128), lambda g: (g, 0))] * 4,
      out_shape=[jax.ShapeDtypeStruct((npad, 128), jnp.float32)] * 4,
  )(praw, xp128, w1e, b1t)


def _tc_out(qflat, hps, xp128, w2, b2r, wiht, bihr, whht, bhhr, wfct, bfcr,
            n, npad, t_steps, h_gcn, h_gru):
  nb = npad // _RB
  npb = npad // _RB

  def body(q0, q1, q2, q3, h0, h1, h2, h3, xref, w2ref, b2ref, wihref,
           bihref, whhref, bhhref, wfcref, bfcref, pooled_ref, out_ref):
    g = pl.program_id(0)
    dinv = xref[:, 24:25]
    qr = jnp.concatenate([q0[...], q1[...], q2[...], q3[...]], axis=1)
    hp = jnp.concatenate([h0[...], h1[...], h2[...], h3[...]], axis=1)
    q = dinv * (qr + hp)
    rowid = g * _RB + lax.broadcasted_iota(jnp.int32, (_RB, h_gcn), 0)
    mask = jnp.where(rowid < n, 1.0, 0.0)
    parts = []
    for t in range(t_steps):
      zt = jnp.maximum(
          jnp.dot(q[:, t * h_gcn:(t + 1) * h_gcn], w2ref[...],
                  preferred_element_type=jnp.float32) + b2ref[...], 0.0)
      parts.append(jnp.sum(zt * mask, axis=0, keepdims=True))
    contrib = jnp.concatenate(parts, axis=0)

    @pl.when(g == 0)
    def _():
      pooled_ref[...] = jnp.zeros_like(pooled_ref)
      out_ref[...] = jnp.zeros_like(out_ref)

    pooled_ref[...] += contrib

    @pl.when(g == nb - 1)
    def _():
      seq = pooled_ref[...] * (1.0 / n)
      h = jnp.zeros((1, h_gru), jnp.float32)
      for t in range(t_steps):
        st = seq[t:t + 1, :]
        gx = jnp.dot(st, wihref[...],
                     preferred_element_type=jnp.float32) + bihref[...]
        gh = jnp.dot(h, whhref[...],
                     preferred_element_type=jnp.float32) + bhhref[...]
        r = jax.nn.sigmoid(gx[:, :h_gru] + gh[:, :h_gru])
        z = jax.nn.sigmoid(gx[:, h_gru:2 * h_gru] + gh[:, h_gru:2 * h_gru])
        nn = jnp.tanh(gx[:, 2 * h_gru:] + r * gh[:, 2 * h_gru:])
        h = (1.0 - z) * nn + z * h
      out_ref[...] = jnp.dot(h, wfcref[...],
                             preferred_element_type=jnp.float32) + bfcref[...]

  qspecs = [pl.BlockSpec((_RB, 128), functools.partial(
      lambda g, kk: (kk * npb + g, 0), kk=kk)) for kk in range(4)]
  hspecs = [pl.BlockSpec((_RB, 128), lambda g: (g, 0)) for _ in range(4)]
  pooled, out = pl.pallas_call(
      body,
      grid=(nb,),
      in_specs=qspecs + hspecs + [
          pl.BlockSpec((_RB, 128), lambda g: (g, 0)),
          pl.BlockSpec((h_gcn, h_gcn), lambda g: (0, 0)),
          pl.BlockSpec((1, h_gcn), lambda g: (0, 0)),
          pl.BlockSpec((h_gcn, 3 * h_gru), lambda g: (0, 0)),
          pl.BlockSpec((1, 3 * h_gru), lambda g: (0, 0)),
          pl.BlockSpec((h_gru, 3 * h_gru), lambda g: (0, 0)),
          pl.BlockSpec((1, 3 * h_gru), lambda g: (0, 0)),
          pl.BlockSpec((h_gru, 128), lambda g: (0, 0)),
          pl.BlockSpec((1, 128), lambda g: (0, 0)),
      ],
      out_specs=[
          pl.BlockSpec((t_steps, h_gcn), lambda g: (0, 0)),
          pl.BlockSpec((1, 128), lambda g: (0, 0)),
      ],
      out_shape=[
          jax.ShapeDtypeStruct((t_steps, h_gcn), jnp.float32),
          jax.ShapeDtypeStruct((1, 128), jnp.float32),
      ],
  )(qflat, qflat, qflat, qflat, *hps, xp128, w2, b2r, wiht, bihr, whht,
    bhhr, wfct, bfcr)
  del pooled
  return out


def kernel(x, edge_index, W1, b1, W2, b2, Wih, Whh, bih, bhh, Wfc, bfc):
  t_steps, n, d_in = x.shape
  h_gcn = W1.shape[1]
  h_gru = Whh.shape[1]
  hw = t_steps * h_gcn
  npad = ((n + _RB - 1) // _RB) * _RB
  src, dst = edge_index[0], edge_index[1]

  xt = jnp.transpose(x, (1, 0, 2)).reshape(n, t_steps * d_in)
  xt128 = jnp.zeros((npad, 128), jnp.float32)
  xt128 = xt128.at[:n, :t_steps * d_in].set(xt)
  xt128 = xt128.at[:n, 24].set(1.0)

  w1e = jnp.zeros((128, hw), jnp.float32)
  for t in range(t_steps):
    w1e = w1e.at[t * d_in:(t + 1) * d_in, t * h_gcn:(t + 1) * h_gcn].set(W1)
  b1t = jnp.tile(b1, (t_steps,)).reshape(1, hw)
  b2r = b2.reshape(1, h_gcn)
  wiht = Wih.T
  whht = Whh.T
  bihr = bih.reshape(1, 3 * h_gru)
  bhhr = bhh.reshape(1, 3 * h_gru)
  wfct = jnp.zeros((h_gru, 128), jnp.float32).at[:, :1].set(Wfc.T)
  bfcr = jnp.zeros((1, 128), jnp.float32).at[0, 0].set(bfc[0])

  deg128 = _sc_pass(src, dst, [], npad)
  xp128 = _tc_norm_scale(deg128, xt128, npad)
  praw = _sc_pass(src, dst, [xp128], npad)
  hps = _tc_hidden(praw, xp128, w1e, b1t, npad, hw)
  qflat = _sc_pass(src, dst, list(hps), npad)
  out = _tc_out(qflat, hps, xp128, W2, b2r, wiht, bihr, whht, bhhr, wfct,
                bfcr, n, npad, t_steps, h_gcn, h_gru)
  return out[0, 0]
